# Initial kernel scaffold; baseline (speedup 1.0000x reference)
import jax
import jax.numpy as jnp
from jax.experimental import pallas as pl


def kernel(inputs_flatten, dictionary):
    raise NotImplementedError("write your pallas kernel here")


# speedup vs baseline: 11.4814x; 11.4814x over previous
import functools

import jax
import jax.numpy as jnp
from jax.experimental import pallas as pl

TOPK = 8
ROW_BLOCK = 256


def _topk_body(x_ref, dic_ref, idx_ref, emb_ref, *, n, k):
    x = x_ref[...]
    dic = dic_ref[...]
    rb, d = x.shape
    x2 = jnp.sum(x * x, axis=1, keepdims=True)
    ones = jnp.ones((1, d), jnp.float32)
    d2 = jax.lax.dot_general(ones, dic * dic,
                             (((1,), (1,)), ((), ())),
                             precision=jax.lax.Precision.HIGHEST,
                             preferred_element_type=jnp.float32)
    mm = jax.lax.dot_general(x.astype(jnp.bfloat16), dic.astype(jnp.bfloat16),
                             (((1,), (1,)), ((), ())),
                             preferred_element_type=jnp.float32)
    s = (x2 + d2) - 2.0 * mm
    iota = jax.lax.broadcasted_iota(jnp.int32, (rb, n), 1)
    mh = jnp.zeros((rb, n), jnp.float32)
    cols = []
    for _ in range(k):
        m = jnp.min(s, axis=1, keepdims=True)
        cand = jnp.where(s == m, iota, n)
        ik = jnp.min(cand, axis=1, keepdims=True)
        cols.append(ik)
        hit = iota == ik
        s = jnp.where(hit, jnp.float32(3.0e38), s)
        mh = mh + hit.astype(jnp.float32)
    idx_ref[...] = jnp.concatenate(cols, axis=1)
    emb_ref[...] = jax.lax.dot_general(
        mh, dic, (((1,), (0,)), ((), ())),
        precision=jax.lax.Precision.HIGHEST,
        preferred_element_type=jnp.float32) * (1.0 / k)


@jax.jit
def kernel(inputs_flatten, dictionary):
    b, d = inputs_flatten.shape
    n, _ = dictionary.shape
    rb = min(ROW_BLOCK, b)
    idx, emb = pl.pallas_call(
        functools.partial(_topk_body, n=n, k=TOPK),
        grid=(b // rb,),
        in_specs=[
            pl.BlockSpec((rb, d), lambda i: (i, 0)),
            pl.BlockSpec((n, d), lambda i: (0, 0)),
        ],
        out_specs=[
            pl.BlockSpec((rb, TOPK), lambda i: (i, 0)),
            pl.BlockSpec((rb, d), lambda i: (i, 0)),
        ],
        out_shape=[
            jax.ShapeDtypeStruct((b, TOPK), jnp.int32),
            jax.ShapeDtypeStruct((b, d), jnp.float32),
        ],
    )(inputs_flatten, dictionary)
    return (emb, idx)


# speedup vs baseline: 24.7589x; 2.1564x over previous
import functools

import jax
import jax.numpy as jnp
from jax import lax
from jax.experimental import pallas as pl
from jax.experimental.pallas import tpu as pltpu
from jax.experimental.pallas import tpu_sc as plsc

TOPK = 8
ROW_BLOCK = 256


def _topk_body(x_ref, dic_ref, idx_ref, *, n, k):
    x = x_ref[...]
    dic = dic_ref[...]
    rb, d = x.shape
    x2 = jnp.sum(x * x, axis=1, keepdims=True)
    ones = jnp.ones((1, d), jnp.float32)
    d2 = jax.lax.dot_general(ones, dic * dic,
                             (((1,), (1,)), ((), ())),
                             precision=jax.lax.Precision.HIGHEST,
                             preferred_element_type=jnp.float32)
    mm = jax.lax.dot_general(x.astype(jnp.bfloat16), dic.astype(jnp.bfloat16),
                             (((1,), (1,)), ((), ())),
                             preferred_element_type=jnp.float32)
    s = (x2 + d2) - 2.0 * mm
    iota = jax.lax.broadcasted_iota(jnp.int32, (rb, n), 1)
    cols = []
    for _ in range(k):
        m = jnp.min(s, axis=1, keepdims=True)
        cand = jnp.where(s == m, iota, n)
        ik = jnp.min(cand, axis=1, keepdims=True)
        cols.append(ik)
        s = jnp.where(iota == ik, jnp.float32(3.0e38), s)
    idx_ref[...] = jnp.concatenate(cols, axis=1)


def _topk_indices(inputs_flatten, dictionary):
    b, d = inputs_flatten.shape
    n, _ = dictionary.shape
    rb = min(ROW_BLOCK, b)
    return pl.pallas_call(
        functools.partial(_topk_body, n=n, k=TOPK),
        grid=(b // rb,),
        in_specs=[
            pl.BlockSpec((rb, d), lambda i: (i, 0)),
            pl.BlockSpec((n, d), lambda i: (0, 0)),
        ],
        out_specs=pl.BlockSpec((rb, TOPK), lambda i: (i, 0)),
        out_shape=jax.ShapeDtypeStruct((b, TOPK), jnp.int32),
    )(inputs_flatten, dictionary)


def _sc_gather_mean(dictionary, idx_flat, b, d, k):
    info = plsc.get_sparse_core_info()
    nw = info.num_cores * info.num_subcores
    rows_w = b // nw
    chunk = 32
    n_chunks = rows_w // chunk
    mesh = plsc.VectorSubcoreMesh(core_axis_name="c", subcore_axis_name="s")

    @functools.partial(
        pl.kernel,
        mesh=mesh,
        out_type=jax.ShapeDtypeStruct((b, d), jnp.float32),
        scratch_types=[
            pltpu.VMEM((chunk * k,), jnp.int32),
            pltpu.VMEM((chunk * k, d), jnp.float32),
            pltpu.VMEM((chunk, d), jnp.float32),
            pltpu.SemaphoreType.DMA,
        ],
    )
    def gather_mean(dic_hbm, idx_hbm, out_hbm, idx_v, rows_v, out_v, sem):
        wid = lax.axis_index("s") * info.num_cores + lax.axis_index("c")
        row0 = wid * rows_w

        def do_chunk(c, _):
            base = row0 + c * chunk
            pltpu.sync_copy(idx_hbm.at[pl.ds(base * k, chunk * k)], idx_v)
            pltpu.async_copy(dic_hbm.at[idx_v], rows_v, sem).wait()

            def accum(r, _):
                r8 = r * k
                for cc in range(d // 16):
                    sl = pl.ds(cc * 16, 16)
                    acc = rows_v[r8, sl]
                    for kk in range(1, k):
                        acc = acc + rows_v[r8 + kk, sl]
                    out_v[r, sl] = acc * (1.0 / k)
                return 0

            lax.fori_loop(0, chunk, accum, 0)
            pltpu.sync_copy(out_v, out_hbm.at[pl.ds(base, chunk)])
            return 0

        lax.fori_loop(0, n_chunks, do_chunk, 0)

    return gather_mean(dictionary, idx_flat)


@jax.jit
def kernel(inputs_flatten, dictionary):
    b, d = inputs_flatten.shape
    idx = _topk_indices(inputs_flatten, dictionary)
    emb = _sc_gather_mean(dictionary, idx.reshape(-1), b, d, TOPK)
    return (emb, idx)


# speedup vs baseline: 31.5776x; 1.2754x over previous
import functools

import jax
import jax.numpy as jnp
from jax import lax
from jax.experimental import pallas as pl
from jax.experimental.pallas import tpu as pltpu
from jax.experimental.pallas import tpu_sc as plsc

TOPK = 8
ROW_BLOCK = 256


def _topk_body(x_ref, dic_ref, idx_ref, d2_ref, *, n, k):
    x = x_ref[...]
    dic = dic_ref[...]
    rb, d = x.shape
    x2 = jnp.sum(x * x, axis=1, keepdims=True)

    @pl.when(pl.program_id(0) == 0)
    def _():
        ones = jnp.ones((1, d), jnp.float32)
        d2_ref[...] = jax.lax.dot_general(
            ones, dic * dic, (((1,), (1,)), ((), ())),
            precision=jax.lax.Precision.HIGHEST,
            preferred_element_type=jnp.float32)

    mm = jax.lax.dot_general(x.astype(jnp.bfloat16), dic.astype(jnp.bfloat16),
                             (((1,), (1,)), ((), ())),
                             preferred_element_type=jnp.float32)
    s = (x2 + d2_ref[...]) - 2.0 * mm
    iota = jax.lax.broadcasted_iota(jnp.int32, (rb, n), 1)
    cols = []
    for _ in range(k):
        m = jnp.min(s, axis=1, keepdims=True)
        cand = jnp.where(s == m, iota, n)
        ik = jnp.min(cand, axis=1, keepdims=True)
        cols.append(ik)
        s = jnp.where(iota == ik, jnp.float32(3.0e38), s)
    idx_ref[...] = jnp.concatenate(cols, axis=1)


def _topk_indices(inputs_flatten, dictionary):
    b, d = inputs_flatten.shape
    n, _ = dictionary.shape
    rb = min(ROW_BLOCK, b)
    return pl.pallas_call(
        functools.partial(_topk_body, n=n, k=TOPK),
        grid=(b // rb,),
        in_specs=[
            pl.BlockSpec((rb, d), lambda i: (i, 0)),
            pl.BlockSpec((n, d), lambda i: (0, 0)),
        ],
        out_specs=pl.BlockSpec((rb, TOPK), lambda i: (i, 0)),
        out_shape=jax.ShapeDtypeStruct((b, TOPK), jnp.int32),
        scratch_shapes=[pltpu.VMEM((1, n), jnp.float32)],
    )(inputs_flatten, dictionary)


def _sc_gather_mean(dictionary, idx_flat, b, d, k):
    info = plsc.get_sparse_core_info()
    nw = info.num_cores * info.num_subcores
    rows_w = b // nw
    chunk = 32
    n_chunks = rows_w // chunk
    mesh = plsc.VectorSubcoreMesh(core_axis_name="c", subcore_axis_name="s")

    @functools.partial(
        pl.kernel,
        mesh=mesh,
        out_type=jax.ShapeDtypeStruct((b, d), jnp.float32),
        scratch_types=[
            pltpu.VMEM((chunk * k,), jnp.int32),
            pltpu.VMEM((chunk * k, d), jnp.float32),
            pltpu.VMEM((chunk, d), jnp.float32),
            pltpu.SemaphoreType.DMA,
        ],
    )
    def gather_mean(dic_hbm, idx_hbm, out_hbm, idx_v, rows_v, out_v, sem):
        wid = lax.axis_index("s") * info.num_cores + lax.axis_index("c")
        row0 = wid * rows_w

        def do_chunk(c, _):
            base = row0 + c * chunk
            pltpu.sync_copy(idx_hbm.at[pl.ds(base * k, chunk * k)], idx_v)
            pltpu.async_copy(dic_hbm.at[idx_v], rows_v, sem).wait()

            def accum(r, _):
                r8 = r * k
                for cc in range(d // 16):
                    sl = pl.ds(cc * 16, 16)
                    acc = rows_v[r8, sl]
                    for kk in range(1, k):
                        acc = acc + rows_v[r8 + kk, sl]
                    out_v[r, sl] = acc * (1.0 / k)
                return 0

            lax.fori_loop(0, chunk, accum, 0)
            pltpu.sync_copy(out_v, out_hbm.at[pl.ds(base, chunk)])
            return 0

        lax.fori_loop(0, n_chunks, do_chunk, 0)

    return gather_mean(dictionary, idx_flat)


@jax.jit
def kernel(inputs_flatten, dictionary):
    b, d = inputs_flatten.shape
    idx = _topk_indices(inputs_flatten, dictionary)
    emb = _sc_gather_mean(dictionary, idx.reshape(-1), b, d, TOPK)
    return (emb, idx)


# speedup vs baseline: 33.6612x; 1.0660x over previous
import functools

import jax
import jax.numpy as jnp
from jax import lax
from jax.experimental import pallas as pl
from jax.experimental.pallas import tpu as pltpu
from jax.experimental.pallas import tpu_sc as plsc

TOPK = 8
ROW_BLOCK = 256


def _topk_body(x_ref, dic_ref, idx_ref, d2_ref, *, n, k):
    x = x_ref[...]
    dic = dic_ref[...]
    rb, d = x.shape
    x2 = jnp.sum(x * x, axis=1, keepdims=True)

    @pl.when(pl.program_id(0) == 0)
    def _():
        ones = jnp.ones((1, d), jnp.float32)
        d2_ref[...] = jax.lax.dot_general(
            ones, dic * dic, (((1,), (1,)), ((), ())),
            precision=jax.lax.Precision.HIGHEST,
            preferred_element_type=jnp.float32)

    mm = jax.lax.dot_general(x.astype(jnp.bfloat16), dic.astype(jnp.bfloat16),
                             (((1,), (1,)), ((), ())),
                             preferred_element_type=jnp.float32)
    s = (x2 + d2_ref[...]) - 2.0 * mm
    iota = jax.lax.broadcasted_iota(jnp.int32, (rb, n), 1)
    cols = []
    for _ in range(k):
        ik = jnp.argmin(s, axis=1).astype(jnp.int32).reshape(rb, 1)
        cols.append(ik)
        s = jnp.where(iota == ik, jnp.float32(3.0e38), s)
    idx_ref[...] = jnp.concatenate(cols, axis=1)


def _topk_indices(inputs_flatten, dictionary):
    b, d = inputs_flatten.shape
    n, _ = dictionary.shape
    rb = min(ROW_BLOCK, b)
    return pl.pallas_call(
        functools.partial(_topk_body, n=n, k=TOPK),
        grid=(b // rb,),
        in_specs=[
            pl.BlockSpec((rb, d), lambda i: (i, 0)),
            pl.BlockSpec((n, d), lambda i: (0, 0)),
        ],
        out_specs=pl.BlockSpec((rb, TOPK), lambda i: (i, 0)),
        out_shape=jax.ShapeDtypeStruct((b, TOPK), jnp.int32),
        scratch_shapes=[pltpu.VMEM((1, n), jnp.float32)],
    )(inputs_flatten, dictionary)


def _sc_gather_mean(dictionary, idx_flat, b, d, k):
    info = plsc.get_sparse_core_info()
    nw = info.num_cores * info.num_subcores
    rows_w = b // nw
    chunk = 32
    n_chunks = rows_w // chunk
    mesh = plsc.VectorSubcoreMesh(core_axis_name="c", subcore_axis_name="s")

    @functools.partial(
        pl.kernel,
        mesh=mesh,
        out_type=jax.ShapeDtypeStruct((b, d), jnp.float32),
        scratch_types=[
            pltpu.VMEM((chunk * k,), jnp.int32),
            pltpu.VMEM((chunk * k, d), jnp.float32),
            pltpu.VMEM((chunk, d), jnp.float32),
            pltpu.SemaphoreType.DMA,
        ],
    )
    def gather_mean(dic_hbm, idx_hbm, out_hbm, idx_v, rows_v, out_v, sem):
        wid = lax.axis_index("s") * info.num_cores + lax.axis_index("c")
        row0 = wid * rows_w

        def do_chunk(c, _):
            base = row0 + c * chunk
            pltpu.sync_copy(idx_hbm.at[pl.ds(base * k, chunk * k)], idx_v)
            pltpu.async_copy(dic_hbm.at[idx_v], rows_v, sem).wait()

            def accum(r, _):
                r8 = r * k
                for cc in range(d // 16):
                    sl = pl.ds(cc * 16, 16)
                    acc = rows_v[r8, sl]
                    for kk in range(1, k):
                        acc = acc + rows_v[r8 + kk, sl]
                    out_v[r, sl] = acc * (1.0 / k)
                return 0

            lax.fori_loop(0, chunk, accum, 0)
            pltpu.sync_copy(out_v, out_hbm.at[pl.ds(base, chunk)])
            return 0

        lax.fori_loop(0, n_chunks, do_chunk, 0)

    return gather_mean(dictionary, idx_flat)


@jax.jit
def kernel(inputs_flatten, dictionary):
    b, d = inputs_flatten.shape
    idx = _topk_indices(inputs_flatten, dictionary)
    emb = _sc_gather_mean(dictionary, idx.reshape(-1), b, d, TOPK)
    return (emb, idx)


# speedup vs baseline: 35.4013x; 1.0517x over previous
import functools

import jax
import jax.numpy as jnp
from jax import lax
from jax.experimental import pallas as pl
from jax.experimental.pallas import tpu as pltpu
from jax.experimental.pallas import tpu_sc as plsc

TOPK = 8
ROW_BLOCK = 256


def _topk_body(x_ref, dic_ref, idx_ref, d2_ref, *, n, k):
    x = x_ref[...]
    dic = dic_ref[...]
    rb, d = x.shape
    x2 = jnp.sum(x * x, axis=1, keepdims=True)

    @pl.when(pl.program_id(0) == 0)
    def _():
        ones = jnp.ones((1, d), jnp.float32)
        d2_ref[...] = jax.lax.dot_general(
            ones, dic * dic, (((1,), (1,)), ((), ())),
            precision=jax.lax.Precision.HIGHEST,
            preferred_element_type=jnp.float32)

    mm = jax.lax.dot_general(x.astype(jnp.bfloat16), dic.astype(jnp.bfloat16),
                             (((1,), (1,)), ((), ())),
                             preferred_element_type=jnp.float32)
    s = (x2 + d2_ref[...]) - 2.0 * mm
    iota = jax.lax.broadcasted_iota(jnp.int32, (rb, n), 1)
    cols = []
    for _ in range(k):
        ik = jnp.argmin(s, axis=1).astype(jnp.int32).reshape(rb, 1)
        cols.append(ik)
        s = jnp.where(iota == ik, jnp.float32(3.0e38), s)
    idx_ref[...] = jnp.concatenate(cols, axis=1)


def _topk_indices(inputs_flatten, dictionary):
    b, d = inputs_flatten.shape
    n, _ = dictionary.shape
    rb = min(ROW_BLOCK, b)
    return pl.pallas_call(
        functools.partial(_topk_body, n=n, k=TOPK),
        grid=(b // rb,),
        in_specs=[
            pl.BlockSpec((rb, d), lambda i: (i, 0)),
            pl.BlockSpec((n, d), lambda i: (0, 0)),
        ],
        out_specs=pl.BlockSpec((rb, TOPK), lambda i: (i, 0)),
        out_shape=jax.ShapeDtypeStruct((b, TOPK), jnp.int32),
        scratch_shapes=[pltpu.VMEM((1, n), jnp.float32)],
    )(inputs_flatten, dictionary)


def _sc_gather_mean(dictionary, idx_flat, b, d, k):
    info = plsc.get_sparse_core_info()
    nw = info.num_cores * info.num_subcores
    rows_w = b // nw
    chunk = 16
    n_chunks = rows_w // chunk
    mesh = plsc.VectorSubcoreMesh(core_axis_name="c", subcore_axis_name="s")

    @functools.partial(
        pl.kernel,
        mesh=mesh,
        out_type=jax.ShapeDtypeStruct((b, d), jnp.float32),
        scratch_types=[
            pltpu.VMEM((rows_w * k,), jnp.int32),
            pltpu.VMEM((chunk * k, d), jnp.float32),
            pltpu.VMEM((chunk * k, d), jnp.float32),
            pltpu.VMEM((chunk, d), jnp.float32),
            pltpu.SemaphoreType.DMA,
            pltpu.SemaphoreType.DMA,
        ],
    )
    def gather_mean(dic_hbm, idx_hbm, out_hbm, idx_v, rows_a, rows_b,
                    out_v, sem_a, sem_b):
        wid = lax.axis_index("s") * info.num_cores + lax.axis_index("c")
        row0 = wid * rows_w
        pltpu.sync_copy(idx_hbm.at[pl.ds(row0 * k, rows_w * k)], idx_v)
        bufs = ((rows_a, sem_a), (rows_b, sem_b))

        def gather(c, buf, sem):
            return pltpu.async_copy(
                dic_hbm.at[idx_v.at[pl.ds(c * (chunk * k), chunk * k)]],
                buf, sem)

        gather(0, rows_a, sem_a)
        gather(1, rows_b, sem_b)

        def pair(g, _):
            for bb in range(2):
                buf, sem = bufs[bb]
                c = 2 * g + bb
                pltpu.make_async_copy(
                    dic_hbm.at[idx_v.at[pl.ds(c * (chunk * k), chunk * k)]],
                    buf, sem).wait()

                def accum(r, _):
                    r8 = r * k
                    for cc in range(d // 16):
                        sl = pl.ds(cc * 16, 16)
                        acc = buf[r8, sl]
                        for kk in range(1, k):
                            acc = acc + buf[r8 + kk, sl]
                        out_v[r, sl] = acc * (1.0 / k)
                    return 0

                lax.fori_loop(0, chunk, accum, 0)
                pltpu.sync_copy(out_v,
                                out_hbm.at[pl.ds(row0 + c * chunk, chunk)])

                @pl.when(c + 2 < n_chunks)
                def _():
                    gather(c + 2, buf, sem)
            return 0

        lax.fori_loop(0, n_chunks // 2, pair, 0)

    return gather_mean(dictionary, idx_flat)


@jax.jit
def kernel(inputs_flatten, dictionary):
    b, d = inputs_flatten.shape
    idx = _topk_indices(inputs_flatten, dictionary)
    emb = _sc_gather_mean(dictionary, idx.reshape(-1), b, d, TOPK)
    return (emb, idx)
